# initial kernel scaffold (unmeasured)
import jax
import jax.numpy as jnp
from jax import lax
from jax.experimental import pallas as pl
from jax.experimental.pallas import tpu as pltpu


def kernel(
    x,
):
    def body(*refs):
        pass

    out_shape = jax.ShapeDtypeStruct(..., jnp.float32)
    return pl.pallas_call(body, out_shape=out_shape)(...)


# baseline (device time: 57976 ns/iter reference)
import jax
import jax.numpy as jnp
from jax import lax
from jax.experimental import pallas as pl
from jax.experimental.pallas import tpu as pltpu

M_PER = 2048
N_PER = 512
HALF = 1024


def kernel(x):
    m, n = x.shape

    def body(x_ref, out_ref, send_sems, recv_sems):
        my_x = lax.axis_index("x")
        my_y = lax.axis_index("y")
        other_x = 1 - my_x
        other_y = 1 - my_y

        barrier_sem = pltpu.get_barrier_semaphore()
        pl.semaphore_signal(barrier_sem, inc=1, device_id=(my_x, other_y),
                            device_id_type=pl.DeviceIdType.MESH)
        pl.semaphore_signal(barrier_sem, inc=1, device_id=(other_x, my_y),
                            device_id_type=pl.DeviceIdType.MESH)
        pl.semaphore_wait(barrier_sem, 2)

        out_ref[pl.ds(my_y * M_PER, M_PER), :] = x_ref[:, pl.ds(my_y * N_PER, N_PER)]

        row_off = my_y * M_PER + my_x * HALF
        rdma1 = pltpu.make_async_remote_copy(
            src_ref=x_ref.at[pl.ds(my_x * HALF, HALF), pl.ds(other_y * N_PER, N_PER)],
            dst_ref=out_ref.at[pl.ds(row_off, HALF), :],
            send_sem=send_sems.at[0],
            recv_sem=recv_sems.at[0],
            device_id=(my_x, other_y),
            device_id_type=pl.DeviceIdType.MESH,
        )
        rdma1.start()
        rdma1.wait()

        recv_off = other_y * M_PER + my_x * HALF
        rdma2 = pltpu.make_async_remote_copy(
            src_ref=out_ref.at[pl.ds(recv_off, HALF), :],
            dst_ref=out_ref.at[pl.ds(recv_off, HALF), :],
            send_sem=send_sems.at[1],
            recv_sem=recv_sems.at[1],
            device_id=(other_x, my_y),
            device_id_type=pl.DeviceIdType.MESH,
        )
        rdma2.start()
        rdma2.wait()

    return pl.pallas_call(
        body,
        out_shape=jax.ShapeDtypeStruct((2 * m, n // 2), x.dtype),
        in_specs=[pl.BlockSpec(memory_space=pltpu.VMEM)],
        out_specs=pl.BlockSpec(memory_space=pltpu.VMEM),
        scratch_shapes=[
            pltpu.SemaphoreType.DMA((2,)),
            pltpu.SemaphoreType.DMA((2,)),
        ],
        compiler_params=pltpu.CompilerParams(collective_id=0),
    )(x)


# device time: 37218 ns/iter; 1.5577x vs baseline; 1.5577x over previous
import jax
import jax.numpy as jnp
from jax import lax
from jax.experimental import pallas as pl
from jax.experimental.pallas import tpu as pltpu

M_PER = 2048
N_PER = 512
HALF = 1024
K = 16
CH = HALF // K


def kernel(x):
    m, n = x.shape

    def body(x_ref, out_ref, y_send_sems, y_recv_sems, x_send_sems,
             x_recv_sems, local_sem):
        my_x = lax.axis_index("x")
        my_y = lax.axis_index("y")
        other_x = 1 - my_x
        other_y = 1 - my_y

        barrier_sem = pltpu.get_barrier_semaphore()
        pl.semaphore_signal(barrier_sem, inc=1, device_id=(my_x, other_y),
                            device_id_type=pl.DeviceIdType.MESH)
        pl.semaphore_signal(barrier_sem, inc=1, device_id=(other_x, my_y),
                            device_id_type=pl.DeviceIdType.MESH)
        pl.semaphore_wait(barrier_sem, 2)

        send_off = my_x * HALF
        dst_off = my_y * M_PER + my_x * HALF
        y_rdmas = []
        for k in range(K):
            r = pltpu.make_async_remote_copy(
                src_ref=x_ref.at[pl.ds(send_off + k * CH, CH),
                                 pl.ds(other_y * N_PER, N_PER)],
                dst_ref=out_ref.at[pl.ds(dst_off + k * CH, CH), :],
                send_sem=y_send_sems.at[k],
                recv_sem=y_recv_sems.at[k],
                device_id=(my_x, other_y),
                device_id_type=pl.DeviceIdType.MESH,
            )
            r.start()
            y_rdmas.append(r)

        local_copy = pltpu.make_async_copy(
            x_ref.at[:, pl.ds(my_y * N_PER, N_PER)],
            out_ref.at[pl.ds(my_y * M_PER, M_PER), :],
            local_sem,
        )
        local_copy.start()

        recv_off = other_y * M_PER + my_x * HALF
        x_rdmas = []
        for k in range(K):
            y_rdmas[k].wait_recv()
            r = pltpu.make_async_remote_copy(
                src_ref=out_ref.at[pl.ds(recv_off + k * CH, CH), :],
                dst_ref=out_ref.at[pl.ds(recv_off + k * CH, CH), :],
                send_sem=x_send_sems.at[k],
                recv_sem=x_recv_sems.at[k],
                device_id=(other_x, my_y),
                device_id_type=pl.DeviceIdType.MESH,
            )
            r.start()
            x_rdmas.append(r)

        for k in range(K):
            y_rdmas[k].wait_send()
            x_rdmas[k].wait()
        local_copy.wait()

    return pl.pallas_call(
        body,
        out_shape=jax.ShapeDtypeStruct((2 * m, n // 2), x.dtype),
        in_specs=[pl.BlockSpec(memory_space=pltpu.VMEM)],
        out_specs=pl.BlockSpec(memory_space=pltpu.VMEM),
        scratch_shapes=[
            pltpu.SemaphoreType.DMA((K,)),
            pltpu.SemaphoreType.DMA((K,)),
            pltpu.SemaphoreType.DMA((K,)),
            pltpu.SemaphoreType.DMA((K,)),
            pltpu.SemaphoreType.DMA,
        ],
        compiler_params=pltpu.CompilerParams(collective_id=0),
    )(x)


# device time: 37067 ns/iter; 1.5641x vs baseline; 1.0041x over previous
import jax
import jax.numpy as jnp
from jax import lax
from jax.experimental import pallas as pl
from jax.experimental.pallas import tpu as pltpu

M_PER = 2048
N_PER = 512
HALF = 1024
K = 32
CH = HALF // K


def kernel(x):
    m, n = x.shape

    def body(x_ref, out_ref, y_send_sems, y_recv_sems, x_send_sems,
             x_recv_sems, local_sem):
        my_x = lax.axis_index("x")
        my_y = lax.axis_index("y")
        other_x = 1 - my_x
        other_y = 1 - my_y

        barrier_sem = pltpu.get_barrier_semaphore()
        pl.semaphore_signal(barrier_sem, inc=1, device_id=(my_x, other_y),
                            device_id_type=pl.DeviceIdType.MESH)
        pl.semaphore_signal(barrier_sem, inc=1, device_id=(other_x, my_y),
                            device_id_type=pl.DeviceIdType.MESH)
        pl.semaphore_wait(barrier_sem, 2)

        send_off = my_x * HALF
        dst_off = my_y * M_PER + my_x * HALF
        y_rdmas = []
        for k in range(K):
            r = pltpu.make_async_remote_copy(
                src_ref=x_ref.at[pl.ds(send_off + k * CH, CH),
                                 pl.ds(other_y * N_PER, N_PER)],
                dst_ref=out_ref.at[pl.ds(dst_off + k * CH, CH), :],
                send_sem=y_send_sems.at[k],
                recv_sem=y_recv_sems.at[k],
                device_id=(my_x, other_y),
                device_id_type=pl.DeviceIdType.MESH,
            )
            r.start()
            y_rdmas.append(r)

        local_copy = pltpu.make_async_copy(
            x_ref.at[:, pl.ds(my_y * N_PER, N_PER)],
            out_ref.at[pl.ds(my_y * M_PER, M_PER), :],
            local_sem,
        )
        local_copy.start()

        recv_off = other_y * M_PER + my_x * HALF
        x_rdmas = []
        for k in range(K):
            y_rdmas[k].wait_recv()
            r = pltpu.make_async_remote_copy(
                src_ref=out_ref.at[pl.ds(recv_off + k * CH, CH), :],
                dst_ref=out_ref.at[pl.ds(recv_off + k * CH, CH), :],
                send_sem=x_send_sems.at[k],
                recv_sem=x_recv_sems.at[k],
                device_id=(other_x, my_y),
                device_id_type=pl.DeviceIdType.MESH,
            )
            r.start()
            x_rdmas.append(r)

        for k in range(K):
            y_rdmas[k].wait_send()
            x_rdmas[k].wait()
        local_copy.wait()

    return pl.pallas_call(
        body,
        out_shape=jax.ShapeDtypeStruct((2 * m, n // 2), x.dtype),
        in_specs=[pl.BlockSpec(memory_space=pltpu.VMEM)],
        out_specs=pl.BlockSpec(memory_space=pltpu.VMEM),
        scratch_shapes=[
            pltpu.SemaphoreType.DMA((K,)),
            pltpu.SemaphoreType.DMA((K,)),
            pltpu.SemaphoreType.DMA((K,)),
            pltpu.SemaphoreType.DMA((K,)),
            pltpu.SemaphoreType.DMA,
        ],
        compiler_params=pltpu.CompilerParams(collective_id=0),
    )(x)
